# baseline (speedup 1.0000x reference)
import jax
import jax.numpy as jnp
from jax.experimental import pallas as pl
from jax.experimental.pallas import tpu as pltpu

_N_SLOTS = 64
_EMBED_DIM = 256
_N_CODES = 8192
_BETA = 0.25
_CHUNK = 1024
_N_CHUNKS = _N_CODES // _CHUNK


def _fused_vq_kernel(slots_ref, W_enc_ref, b_enc_ref, W_prev_ref, b_prev_ref,
                     cb_ref, W_post_ref, b_post_ref, W_dec_ref, b_dec_ref,
                     rec_ref, loss_ref, idx_ref,
                     s_sc, a_sc, rund_sc, runi_sc, runz_sc):
    i = pl.program_id(0)
    f32 = jnp.float32

    @pl.when(i == 0)
    def _encode():
        h = jnp.maximum(
            jnp.dot(slots_ref[...], W_enc_ref[...], preferred_element_type=f32)
            + b_enc_ref[...], 0.0)
        s0 = (jnp.dot(h, W_prev_ref[...], preferred_element_type=f32)
              + b_prev_ref[...])
        s_sc[...] = s0
        a_sc[...] = jnp.sum(s0 * s0, axis=1, keepdims=True)

    s = s_sc[...]
    a = a_sc[...]
    cb = cb_ref[...]
    m = jax.lax.dot_general(s, cb, (((1,), (1,)), ((), ())),
                            preferred_element_type=f32)
    cn = jnp.sum(cb * cb, axis=1)
    d = (a - 2.0 * m) + cn[None, :]
    dmin = jnp.min(d, axis=1, keepdims=True)
    col = jax.lax.broadcasted_iota(jnp.int32, d.shape, 1) + i * _CHUNK
    big = jnp.int32(jnp.iinfo(jnp.int32).max)
    lidx = jnp.min(jnp.where(d == dmin, col, big), axis=1)
    onehot = (col == lidx[:, None]).astype(f32)
    lzq = jnp.dot(onehot, cb, preferred_element_type=f32)

    @pl.when(i == 0)
    def _first():
        rund_sc[...] = dmin
        runi_sc[...] = lidx[:, None]
        runz_sc[...] = lzq

    @pl.when(i > 0)
    def _merge():
        better = dmin < rund_sc[...]
        rund_sc[...] = jnp.where(better, dmin, rund_sc[...])
        runi_sc[...] = jnp.where(better, lidx[:, None], runi_sc[...])
        runz_sc[...] = jnp.where(better, lzq, runz_sc[...])

    @pl.when(i == _N_CHUNKS - 1)
    def _decode():
        zq = runz_sc[...]
        diff = zq - s
        loss = (1.0 + _BETA) * jnp.sum(diff * diff) / (_N_SLOTS * _EMBED_DIM)
        loss_ref[...] = jnp.reshape(loss, (1, 1))
        dec_in = (jnp.dot(zq, W_post_ref[...], preferred_element_type=f32)
                  + b_post_ref[...])
        rec = (jnp.dot(dec_in, W_dec_ref[...], preferred_element_type=f32)
               + b_dec_ref[...])
        rec_ref[...] = jnp.clip(rec, -1.0, 1.0)
        idx_ref[...] = runi_sc[...]


def kernel(img, targets, slots, W_enc, b_enc, W_prev, b_prev, codebook,
           W_post, b_post, W_dec, b_dec):
    bs = img.shape[0]
    enc_dim = W_dec.shape[1]
    f32 = jnp.float32
    whole = lambda shape: pl.BlockSpec(shape, lambda i: (0, 0))
    rec1, loss, idx = pl.pallas_call(
        _fused_vq_kernel,
        grid=(_N_CHUNKS,),
        in_specs=[
            whole((_N_SLOTS, W_enc.shape[0])),
            whole(W_enc.shape),
            whole((1, b_enc.shape[0])),
            whole(W_prev.shape),
            whole((1, b_prev.shape[0])),
            pl.BlockSpec((_CHUNK, _EMBED_DIM), lambda i: (i, 0)),
            whole(W_post.shape),
            whole((1, b_post.shape[0])),
            whole(W_dec.shape),
            whole((1, b_dec.shape[0])),
        ],
        out_specs=[
            whole((_N_SLOTS, enc_dim)),
            whole((1, 1)),
            whole((_N_SLOTS, 1)),
        ],
        out_shape=[
            jax.ShapeDtypeStruct((_N_SLOTS, enc_dim), f32),
            jax.ShapeDtypeStruct((1, 1), f32),
            jax.ShapeDtypeStruct((_N_SLOTS, 1), jnp.int32),
        ],
        scratch_shapes=[
            pltpu.VMEM((_N_SLOTS, _EMBED_DIM), f32),
            pltpu.VMEM((_N_SLOTS, 1), f32),
            pltpu.VMEM((_N_SLOTS, 1), f32),
            pltpu.VMEM((_N_SLOTS, 1), jnp.int32),
            pltpu.VMEM((_N_SLOTS, _EMBED_DIM), f32),
        ],
    )(slots, W_enc, b_enc.reshape(1, -1), W_prev, b_prev.reshape(1, -1),
      codebook, W_post, b_post.reshape(1, -1), W_dec, b_dec.reshape(1, -1))
    rec = jnp.broadcast_to(rec1[None], (bs, _N_SLOTS, enc_dim))
    q_indices = jnp.broadcast_to(idx.reshape(1, _N_SLOTS), (bs, _N_SLOTS))
    return rec, jnp.reshape(loss, ()), q_indices


# speedup vs baseline: 1.2021x; 1.2021x over previous
import jax
import jax.numpy as jnp
from jax.experimental import pallas as pl
from jax.experimental.pallas import tpu as pltpu

_N_SLOTS = 64
_EMBED_DIM = 256
_N_CODES = 8192
_BETA = 0.25
_CHUNK = 2048
_N_CHUNKS = _N_CODES // _CHUNK


def _fused_vq_kernel(slots_ref, W_enc_ref, b_enc_ref, W_prev_ref, b_prev_ref,
                     cb_ref, W_post_ref, b_post_ref, W_dec_ref, b_dec_ref,
                     rec_ref, loss_ref, idx_ref,
                     s_sc, a_sc, rund_sc, runi_sc, runz_sc):
    i = pl.program_id(0)
    f32 = jnp.float32

    @pl.when(i == 0)
    def _encode():
        h = jnp.maximum(
            jnp.dot(slots_ref[...], W_enc_ref[...], preferred_element_type=f32)
            + b_enc_ref[...], 0.0)
        s0 = (jnp.dot(h, W_prev_ref[...], preferred_element_type=f32)
              + b_prev_ref[...])
        s_sc[...] = s0
        a_sc[...] = jnp.sum(s0 * s0, axis=1, keepdims=True)

    s = s_sc[...]
    a = a_sc[...]
    cb = cb_ref[...]
    m = jax.lax.dot_general(s, cb, (((1,), (1,)), ((), ())),
                            preferred_element_type=f32)
    cn = jnp.sum(cb * cb, axis=1)
    d = (a - 2.0 * m) + cn[None, :]
    dmin = jnp.min(d, axis=1, keepdims=True)
    col = jax.lax.broadcasted_iota(jnp.int32, d.shape, 1) + i * _CHUNK
    big = jnp.int32(jnp.iinfo(jnp.int32).max)
    lidx = jnp.min(jnp.where(d == dmin, col, big), axis=1)
    onehot = (col == lidx[:, None]).astype(f32)
    lzq = jnp.dot(onehot, cb, preferred_element_type=f32)

    @pl.when(i == 0)
    def _first():
        rund_sc[...] = dmin
        runi_sc[...] = lidx[:, None]
        runz_sc[...] = lzq

    @pl.when(i > 0)
    def _merge():
        better = dmin < rund_sc[...]
        rund_sc[...] = jnp.where(better, dmin, rund_sc[...])
        runi_sc[...] = jnp.where(better, lidx[:, None], runi_sc[...])
        runz_sc[...] = jnp.where(better, lzq, runz_sc[...])

    @pl.when(i == _N_CHUNKS - 1)
    def _decode():
        zq = runz_sc[...]
        diff = zq - s
        loss = (1.0 + _BETA) * jnp.sum(diff * diff) / (_N_SLOTS * _EMBED_DIM)
        loss_ref[...] = jnp.reshape(loss, (1, 1))
        dec_in = (jnp.dot(zq, W_post_ref[...], preferred_element_type=f32)
                  + b_post_ref[...])
        rec = (jnp.dot(dec_in, W_dec_ref[...], preferred_element_type=f32)
               + b_dec_ref[...])
        rec_ref[...] = jnp.clip(rec, -1.0, 1.0)
        idx_ref[...] = runi_sc[...]


def kernel(img, targets, slots, W_enc, b_enc, W_prev, b_prev, codebook,
           W_post, b_post, W_dec, b_dec):
    bs = img.shape[0]
    enc_dim = W_dec.shape[1]
    f32 = jnp.float32
    whole = lambda shape: pl.BlockSpec(shape, lambda i: (0, 0))
    rec1, loss, idx = pl.pallas_call(
        _fused_vq_kernel,
        grid=(_N_CHUNKS,),
        in_specs=[
            whole((_N_SLOTS, W_enc.shape[0])),
            whole(W_enc.shape),
            whole((1, b_enc.shape[0])),
            whole(W_prev.shape),
            whole((1, b_prev.shape[0])),
            pl.BlockSpec((_CHUNK, _EMBED_DIM), lambda i: (i, 0)),
            whole(W_post.shape),
            whole((1, b_post.shape[0])),
            whole(W_dec.shape),
            whole((1, b_dec.shape[0])),
        ],
        out_specs=[
            whole((_N_SLOTS, enc_dim)),
            whole((1, 1)),
            whole((_N_SLOTS, 1)),
        ],
        out_shape=[
            jax.ShapeDtypeStruct((_N_SLOTS, enc_dim), f32),
            jax.ShapeDtypeStruct((1, 1), f32),
            jax.ShapeDtypeStruct((_N_SLOTS, 1), jnp.int32),
        ],
        scratch_shapes=[
            pltpu.VMEM((_N_SLOTS, _EMBED_DIM), f32),
            pltpu.VMEM((_N_SLOTS, 1), f32),
            pltpu.VMEM((_N_SLOTS, 1), f32),
            pltpu.VMEM((_N_SLOTS, 1), jnp.int32),
            pltpu.VMEM((_N_SLOTS, _EMBED_DIM), f32),
        ],
    )(slots, W_enc, b_enc.reshape(1, -1), W_prev, b_prev.reshape(1, -1),
      codebook, W_post, b_post.reshape(1, -1), W_dec, b_dec.reshape(1, -1))
    rec = jnp.broadcast_to(rec1[None], (bs, _N_SLOTS, enc_dim))
    q_indices = jnp.broadcast_to(idx.reshape(1, _N_SLOTS), (bs, _N_SLOTS))
    return rec, jnp.reshape(loss, ()), q_indices
